# baseline (speedup 1.0000x reference)
import jax
import jax.numpy as jnp
from jax.experimental import pallas as pl
from jax.experimental.pallas import tpu as pltpu

P = 16
S = 4096
D = 512
BLK = 512
K = S // BLK
NEG_SLOPE = 0.2
EPS = 1e-16
BF = jnp.bfloat16


def _lrelu(v):
    return jnp.where(v >= 0, v, NEG_SLOPE * v)


def _dot_t(a, bt):
    return jax.lax.dot_general(
        a, bt, (((1,), (1,)), ((), ())), preferred_element_type=jnp.float32)


def _sample_side(hs, hp_bf, asp_row, as_col, ad_col, b):
    e = _lrelu(asp_row + ad_col)
    e_self = _lrelu(as_col + ad_col)
    m = jnp.maximum(jnp.max(e, axis=1, keepdims=True), e_self)
    w = jnp.exp(e - m)
    w_self = jnp.exp(e_self - m)
    denom = jnp.sum(w, axis=1, keepdims=True) + w_self + EPS
    out = (jnp.dot(w.astype(BF), hp_bf, preferred_element_type=jnp.float32)
           + w_self * hs) / denom
    return jnp.maximum(out + b, 0.0)


def _fused_body(fp_ref, fs_ref, w_ref, aa_ref, b_ref, wfc_ref, bfc_ref,
                h_ref, pred_ref,
                g1_s, gp1_s, hp_s, asp_s, adp_s, aspc_s, m_s, s_s, acc_s):
    l = pl.program_id(0)
    k = pl.program_id(1)
    w = w_ref[0]
    asrc_bf = aa_ref[0, 0:1, :]
    aa = aa_ref[0]
    b = b_ref[0]

    @pl.when(l == 0)
    def _layer1():
        @pl.when(k == 0)
        def _():
            hp = jnp.dot(fp_ref[...], w, preferred_element_type=jnp.float32)
            hp_s[...] = hp
            hp_bf = hp.astype(BF)
            asp_s[...] = _dot_t(asrc_bf, hp_bf)
            pcols = _dot_t(hp_bf, aa)
            aspc_s[...] = pcols[:, 0:1]
            adp_s[...] = pcols[:, 1:2]
            m_s[...] = jnp.full_like(m_s, -jnp.inf)
            s_s[...] = jnp.zeros_like(s_s)
            acc_s[...] = jnp.zeros_like(acc_s)

        hp_bf = hp_s[...].astype(BF)
        hs = jnp.dot(fs_ref[...], w, preferred_element_type=jnp.float32)
        hs_bf = hs.astype(BF)
        cols = _dot_t(hs_bf, aa)
        as_col = cols[:, 0:1]
        ad_col = cols[:, 1:2]
        as_row = _dot_t(asrc_bf, hs_bf)

        g = _sample_side(hs, hp_bf, asp_s[...], as_col, ad_col, b)
        g1_s[pl.ds(k * BLK, BLK), :] = g.astype(BF)

        ep = _lrelu(adp_s[...] + as_row)
        new_m = jnp.maximum(m_s[...], jnp.max(ep, axis=1, keepdims=True))
        scale = jnp.exp(m_s[...] - new_m)
        wp = jnp.exp(ep - new_m)
        s_s[...] = s_s[...] * scale + jnp.sum(wp, axis=1, keepdims=True)
        acc_s[...] = (acc_s[...] * scale
                      + jnp.dot(wp.astype(BF), hs_bf,
                                preferred_element_type=jnp.float32))
        m_s[...] = new_m

        @pl.when(k == K - 1)
        def _():
            hp = hp_s[...]
            e_sp = _lrelu(aspc_s[...] + adp_s[...])
            fm = jnp.maximum(m_s[...], e_sp)
            sc = jnp.exp(m_s[...] - fm)
            wsp = jnp.exp(e_sp - fm)
            den = s_s[...] * sc + wsp + EPS
            accf = acc_s[...] * sc + wsp * hp
            gp1_s[...] = jnp.maximum(accf / den + b, 0.0)

    @pl.when(l == 1)
    def _layer2():
        @pl.when(k == 0)
        def _():
            hp = jnp.dot(gp1_s[...].astype(BF), w,
                         preferred_element_type=jnp.float32)
            hp_s[...] = hp
            asp_s[...] = _dot_t(asrc_bf, hp.astype(BF))

        hp_bf = hp_s[...].astype(BF)
        fs_bf = g1_s[pl.ds(k * BLK, BLK), :]
        hs = jnp.dot(fs_bf, w, preferred_element_type=jnp.float32)
        hs_bf = hs.astype(BF)
        cols = _dot_t(hs_bf, aa)
        as_col = cols[:, 0:1]
        ad_col = cols[:, 1:2]

        g = _sample_side(hs, hp_bf, asp_s[...], as_col, ad_col, b)
        h_ref[...] = g
        pred_ref[...] = (jnp.dot(g.astype(BF), wfc_ref[...],
                                 preferred_element_type=jnp.float32)
                         + bfc_ref[...])


@jax.jit
def _run(x, proxies, W1, a_src1, a_dst1, b1, W2, a_src2, a_dst2, b2,
         W_fc, b_fc):
    wstack = jnp.stack([W1, W2]).astype(BF)
    aastack = jnp.stack([jnp.stack([a_src1, a_dst1]),
                         jnp.stack([a_src2, a_dst2])]).astype(BF)
    bstack = jnp.stack([b1[None, :], b2[None, :]])
    C = W_fc.shape[1]
    CP = 128
    wfc = jnp.pad(W_fc, ((0, 0), (0, CP - C))).astype(BF)
    bfc = jnp.pad(b_fc, (0, CP - C))[None, :]

    h2, preds = pl.pallas_call(
        _fused_body,
        grid=(2, K),
        in_specs=[
            pl.BlockSpec((P, D), lambda l, k: (0, 0)),
            pl.BlockSpec((BLK, D), lambda l, k: (k * (1 - l), 0)),
            pl.BlockSpec((1, D, D), lambda l, k: (l, 0, 0)),
            pl.BlockSpec((1, 2, D), lambda l, k: (l, 0, 0)),
            pl.BlockSpec((1, 1, D), lambda l, k: (l, 0, 0)),
            pl.BlockSpec((D, 128), lambda l, k: (0, 0)),
            pl.BlockSpec((1, 128), lambda l, k: (0, 0)),
        ],
        out_specs=[
            pl.BlockSpec((BLK, D), lambda l, k: (k * l, 0)),
            pl.BlockSpec((BLK, 128), lambda l, k: (k * l, 0)),
        ],
        out_shape=[
            jax.ShapeDtypeStruct((S, D), jnp.float32),
            jax.ShapeDtypeStruct((S, 128), jnp.float32),
        ],
        scratch_shapes=[
            pltpu.VMEM((S, D), BF),
            pltpu.VMEM((P, D), jnp.float32),
            pltpu.VMEM((P, D), jnp.float32),
            pltpu.VMEM((1, P), jnp.float32),
            pltpu.VMEM((P, 1), jnp.float32),
            pltpu.VMEM((P, 1), jnp.float32),
            pltpu.VMEM((P, 1), jnp.float32),
            pltpu.VMEM((P, 1), jnp.float32),
            pltpu.VMEM((P, D), jnp.float32),
        ],
    )(proxies.astype(BF), x.astype(BF), wstack, aastack, bstack, wfc, bfc)

    return preds[:, :C], h2


def kernel(x, proxies, W1, a_src1, a_dst1, b1, W2, a_src2, a_dst2, b2,
           W_fc, b_fc):
    return _run(x, proxies, W1, a_src1, a_dst1, b1,
                W2, a_src2, a_dst2, b2, W_fc, b_fc)


# speedup vs baseline: 1.4139x; 1.4139x over previous
import jax
import jax.numpy as jnp
from jax.experimental import pallas as pl
from jax.experimental.pallas import tpu as pltpu

P = 16
S = 4096
D = 512
BLK = 512
K = S // BLK
NEG_SLOPE = 0.2
EPS = 1e-16
BF = jnp.bfloat16


def _lrelu(v):
    return jnp.where(v >= 0, v, NEG_SLOPE * v)


def _row_dot(vec_row, mat):
    return jax.lax.dot_general(
        vec_row, mat, (((1,), (1,)), ((), ())),
        preferred_element_type=jnp.float32)


def _sample_side(hs, hp, asp_row, as_col, ad_col, b):
    e = _lrelu(asp_row + ad_col)
    e_self = _lrelu(as_col + ad_col)
    m = jnp.maximum(jnp.max(e, axis=1, keepdims=True), e_self)
    w = jnp.exp(e - m)
    w_self = jnp.exp(e_self - m)
    denom = jnp.sum(w, axis=1, keepdims=True) + w_self + EPS
    out = (jnp.dot(w, hp, preferred_element_type=jnp.float32)
           + w_self * hs) / denom
    return jnp.maximum(out + b, 0.0)


def _layer1_body(fp_ref, fs_ref, w_ref, asrc_ref, adst_ref, b_ref,
                 gp_ref, gs_ref, m_s, s_s, acc_s):
    k = pl.program_id(0)
    w = w_ref[...]
    asrc = asrc_ref[...]
    adst = adst_ref[...]

    hp = jnp.dot(fp_ref[...], w, preferred_element_type=jnp.float32)
    asp_row = _row_dot(asrc, hp)
    adp_col = jnp.sum(hp * adst, axis=1, keepdims=True)

    hs = jnp.dot(fs_ref[...], w, preferred_element_type=jnp.float32)
    as_col = jnp.sum(hs * asrc, axis=1, keepdims=True)
    ad_col = jnp.sum(hs * adst, axis=1, keepdims=True)
    as_row = _row_dot(asrc, hs)

    gs_ref[...] = _sample_side(hs, hp, asp_row, as_col, ad_col,
                               b_ref[...]).astype(BF)

    @pl.when(k == 0)
    def _():
        m_s[...] = jnp.full_like(m_s, -jnp.inf)
        s_s[...] = jnp.zeros_like(s_s)
        acc_s[...] = jnp.zeros_like(acc_s)

    ep = _lrelu(adp_col + as_row)
    new_m = jnp.maximum(m_s[...], jnp.max(ep, axis=1, keepdims=True))
    scale = jnp.exp(m_s[...] - new_m)
    wp = jnp.exp(ep - new_m)
    s_s[...] = s_s[...] * scale + jnp.sum(wp, axis=1, keepdims=True)
    acc_s[...] = (acc_s[...] * scale
                  + jnp.dot(wp, hs, preferred_element_type=jnp.float32))
    m_s[...] = new_m

    @pl.when(k == K - 1)
    def _():
        asp_col = jnp.sum(hp * asrc, axis=1, keepdims=True)
        e_sp = _lrelu(asp_col + adp_col)
        fm = jnp.maximum(m_s[...], e_sp)
        sc = jnp.exp(m_s[...] - fm)
        wsp = jnp.exp(e_sp - fm)
        den = s_s[...] * sc + wsp + EPS
        accf = acc_s[...] * sc + wsp * hp
        gp_ref[...] = (jnp.maximum(accf / den + b_ref[...], 0.0)).astype(BF)


def _layer2_body(fp_ref, fs_ref, w_ref, asrc_ref, adst_ref, b_ref,
                 wfc_ref, bfc_ref, h_ref, pred_ref):
    w = w_ref[...]
    asrc = asrc_ref[...]
    adst = adst_ref[...]

    hp = jnp.dot(fp_ref[...], w, preferred_element_type=jnp.float32)
    asp_row = _row_dot(asrc, hp)

    hs = jnp.dot(fs_ref[...], w, preferred_element_type=jnp.float32)
    as_col = jnp.sum(hs * asrc, axis=1, keepdims=True)
    ad_col = jnp.sum(hs * adst, axis=1, keepdims=True)

    g = _sample_side(hs, hp, asp_row, as_col, ad_col, b_ref[...])
    h_ref[...] = g
    pred_ref[...] = (jnp.dot(g.astype(BF), wfc_ref[...],
                             preferred_element_type=jnp.float32)
                     + bfc_ref[...])


def _full_spec(shape):
    return pl.BlockSpec(shape, lambda k: (0, 0))


@jax.jit
def _run(x, proxies, W1, a_src1, a_dst1, b1, W2, a_src2, a_dst2, b2,
         W_fc, b_fc):
    as1 = a_src1[None, :]
    ad1 = a_dst1[None, :]
    b1r = b1[None, :]
    as2 = a_src2[None, :]
    ad2 = a_dst2[None, :]
    b2r = b2[None, :]
    C = W_fc.shape[1]
    CP = 128
    wfc = jnp.pad(W_fc, ((0, 0), (0, CP - C))).astype(BF)
    bfc = jnp.pad(b_fc, (0, CP - C))[None, :]

    gp1, gs1 = pl.pallas_call(
        _layer1_body,
        grid=(K,),
        in_specs=[
            _full_spec((P, D)),
            pl.BlockSpec((BLK, D), lambda k: (k, 0)),
            _full_spec((D, D)),
            _full_spec((1, D)),
            _full_spec((1, D)),
            _full_spec((1, D)),
        ],
        out_specs=[
            _full_spec((P, D)),
            pl.BlockSpec((BLK, D), lambda k: (k, 0)),
        ],
        out_shape=[
            jax.ShapeDtypeStruct((P, D), BF),
            jax.ShapeDtypeStruct((S, D), BF),
        ],
        scratch_shapes=[
            pltpu.VMEM((P, 1), jnp.float32),
            pltpu.VMEM((P, 1), jnp.float32),
            pltpu.VMEM((P, D), jnp.float32),
        ],
    )(proxies.astype(BF), x.astype(BF), W1.astype(BF), as1, ad1, b1r)

    h2, preds = pl.pallas_call(
        _layer2_body,
        grid=(K,),
        in_specs=[
            _full_spec((P, D)),
            pl.BlockSpec((BLK, D), lambda k: (k, 0)),
            _full_spec((D, D)),
            _full_spec((1, D)),
            _full_spec((1, D)),
            _full_spec((1, D)),
            _full_spec((D, CP)),
            _full_spec((1, CP)),
        ],
        out_specs=[
            pl.BlockSpec((BLK, D), lambda k: (k, 0)),
            pl.BlockSpec((BLK, CP), lambda k: (k, 0)),
        ],
        out_shape=[
            jax.ShapeDtypeStruct((S, D), jnp.float32),
            jax.ShapeDtypeStruct((S, CP), jnp.float32),
        ],
    )(gp1, gs1, W2.astype(BF), as2, ad2, b2r, wfc, bfc)

    return preds[:, :C], h2


def kernel(x, proxies, W1, a_src1, a_dst1, b1, W2, a_src2, a_dst2, b2,
           W_fc, b_fc):
    return _run(x, proxies, W1, a_src1, a_dst1, b1,
                W2, a_src2, a_dst2, b2, W_fc, b_fc)


# speedup vs baseline: 2.0638x; 1.4597x over previous
import jax
import jax.numpy as jnp
from jax.experimental import pallas as pl
from jax.experimental.pallas import tpu as pltpu

P = 16
S = 4096
D = 512
BLK = 1024
K = S // BLK
NEG_SLOPE = 0.2
EPS = 1e-16


def _lrelu(v):
    return jnp.where(v >= 0, v, NEG_SLOPE * v)


def _row_dot(vec_row, mat):
    return jax.lax.dot_general(
        vec_row, mat, (((1,), (1,)), ((), ())),
        preferred_element_type=jnp.float32)


def _sample_side(hs, hp, asp_row, as_col, ad_col, b):
    e = _lrelu(asp_row + ad_col)
    e_self = _lrelu(as_col + ad_col)
    m = jnp.maximum(jnp.max(e, axis=1, keepdims=True), e_self)
    w = jnp.exp(e - m)
    w_self = jnp.exp(e_self - m)
    denom = jnp.sum(w, axis=1, keepdims=True) + w_self + EPS
    out = (jnp.dot(w, hp, preferred_element_type=jnp.float32)
           + w_self * hs) / denom
    return jnp.maximum(out + b, 0.0)


def _layer1_body(fp_ref, fs_ref, w_ref, asrc_ref, adst_ref, b_ref,
                 gp_ref, gs_ref, m_s, s_s, acc_s):
    k = pl.program_id(0)
    w = w_ref[...]
    asrc = asrc_ref[...]
    adst = adst_ref[...]

    hp = jnp.dot(fp_ref[...], w, preferred_element_type=jnp.float32)
    asp_row = _row_dot(asrc, hp)
    adp_col = jnp.sum(hp * adst, axis=1, keepdims=True)

    hs = jnp.dot(fs_ref[...], w, preferred_element_type=jnp.float32)
    as_col = jnp.sum(hs * asrc, axis=1, keepdims=True)
    ad_col = jnp.sum(hs * adst, axis=1, keepdims=True)
    as_row = _row_dot(asrc, hs)

    gs_ref[...] = _sample_side(hs, hp, asp_row, as_col, ad_col, b_ref[...])

    @pl.when(k == 0)
    def _():
        m_s[...] = jnp.full_like(m_s, -jnp.inf)
        s_s[...] = jnp.zeros_like(s_s)
        acc_s[...] = jnp.zeros_like(acc_s)

    ep = _lrelu(adp_col + as_row)
    new_m = jnp.maximum(m_s[...], jnp.max(ep, axis=1, keepdims=True))
    scale = jnp.exp(m_s[...] - new_m)
    wp = jnp.exp(ep - new_m)
    s_s[...] = s_s[...] * scale + jnp.sum(wp, axis=1, keepdims=True)
    acc_s[...] = (acc_s[...] * scale
                  + jnp.dot(wp, hs, preferred_element_type=jnp.float32))
    m_s[...] = new_m

    @pl.when(k == K - 1)
    def _():
        asp_col = jnp.sum(hp * asrc, axis=1, keepdims=True)
        e_sp = _lrelu(asp_col + adp_col)
        fm = jnp.maximum(m_s[...], e_sp)
        sc = jnp.exp(m_s[...] - fm)
        wsp = jnp.exp(e_sp - fm)
        den = s_s[...] * sc + wsp + EPS
        accf = acc_s[...] * sc + wsp * hp
        gp_ref[...] = jnp.maximum(accf / den + b_ref[...], 0.0)


def _layer2_body(fp_ref, fs_ref, w_ref, asrc_ref, adst_ref, b_ref,
                 wfc_ref, bfc_ref, h_ref, pred_ref):
    w = w_ref[...]
    asrc = asrc_ref[...]
    adst = adst_ref[...]

    hp = jnp.dot(fp_ref[...], w, preferred_element_type=jnp.float32)
    asp_row = _row_dot(asrc, hp)

    hs = jnp.dot(fs_ref[...], w, preferred_element_type=jnp.float32)
    as_col = jnp.sum(hs * asrc, axis=1, keepdims=True)
    ad_col = jnp.sum(hs * adst, axis=1, keepdims=True)

    g = _sample_side(hs, hp, asp_row, as_col, ad_col, b_ref[...])
    h_ref[...] = g
    pred_ref[...] = (jnp.dot(g, wfc_ref[...], preferred_element_type=jnp.float32)
                     + bfc_ref[...])


def _full_spec(shape):
    return pl.BlockSpec(shape, lambda k: (0, 0))


@jax.jit
def _run(x, proxies, W1, a_src1, a_dst1, b1, W2, a_src2, a_dst2, b2,
         W_fc, b_fc):
    as1 = a_src1[None, :]
    ad1 = a_dst1[None, :]
    b1r = b1[None, :]
    as2 = a_src2[None, :]
    ad2 = a_dst2[None, :]
    b2r = b2[None, :]
    C = W_fc.shape[1]
    CP = 128
    wfc = jnp.pad(W_fc, ((0, 0), (0, CP - C)))
    bfc = jnp.pad(b_fc, (0, CP - C))[None, :]

    gp1, gs1 = pl.pallas_call(
        _layer1_body,
        grid=(K,),
        in_specs=[
            _full_spec((P, D)),
            pl.BlockSpec((BLK, D), lambda k: (k, 0)),
            _full_spec((D, D)),
            _full_spec((1, D)),
            _full_spec((1, D)),
            _full_spec((1, D)),
        ],
        out_specs=[
            _full_spec((P, D)),
            pl.BlockSpec((BLK, D), lambda k: (k, 0)),
        ],
        out_shape=[
            jax.ShapeDtypeStruct((P, D), jnp.float32),
            jax.ShapeDtypeStruct((S, D), jnp.float32),
        ],
        scratch_shapes=[
            pltpu.VMEM((P, 1), jnp.float32),
            pltpu.VMEM((P, 1), jnp.float32),
            pltpu.VMEM((P, D), jnp.float32),
        ],
    )(proxies, x, W1, as1, ad1, b1r)

    h2, preds = pl.pallas_call(
        _layer2_body,
        grid=(K,),
        in_specs=[
            _full_spec((P, D)),
            pl.BlockSpec((BLK, D), lambda k: (k, 0)),
            _full_spec((D, D)),
            _full_spec((1, D)),
            _full_spec((1, D)),
            _full_spec((1, D)),
            _full_spec((D, CP)),
            _full_spec((1, CP)),
        ],
        out_specs=[
            pl.BlockSpec((BLK, D), lambda k: (k, 0)),
            pl.BlockSpec((BLK, CP), lambda k: (k, 0)),
        ],
        out_shape=[
            jax.ShapeDtypeStruct((S, D), jnp.float32),
            jax.ShapeDtypeStruct((S, CP), jnp.float32),
        ],
    )(gp1, gs1, W2, as2, ad2, b2r, wfc, bfc)

    return preds[:, :C], h2


def kernel(x, proxies, W1, a_src1, a_dst1, b1, W2, a_src2, a_dst2, b2,
           W_fc, b_fc):
    return _run(x, proxies, W1, a_src1, a_dst1, b1,
                W2, a_src2, a_dst2, b2, W_fc, b_fc)
